# baseline (speedup 1.0000x reference)
import jax
import jax.numpy as jnp
from jax.experimental import pallas as pl
from jax.experimental.pallas import tpu as pltpu

_TN = 256


def _body(w_ref, x_ref, b_ref, bias_ref, mask_ref, o_ref,
          wres_ref, xcur_ref, u_ref):
    b = pl.program_id(0)
    p = pl.program_id(1)
    t = pl.program_id(2)
    nlast = pl.num_programs(1) - 1

    @pl.when(t == 0)
    def _():
        @pl.when(p == 0)
        def _():
            xb = x_ref[0]
            for jj in range(3):
                u_ref[jj] = jnp.dot(xb, b_ref[0, jj],
                                    preferred_element_type=jnp.float32
                                    ).astype(jnp.bfloat16)

        @pl.when(p > 0)
        def _():
            xb = xcur_ref[...]
            for jj in range(3):
                u_ref[jj] = jnp.dot(xb, b_ref[p, jj],
                                    preferred_element_type=jnp.float32
                                    ).astype(jnp.bfloat16)

    rows = pl.ds(t * _TN, _TN)

    def epilogue(z):
        z = z + bias_ref[p][None]
        z16 = jnp.where(p < nlast, jnp.maximum(z[:, :16], 0.0), z[:, :16])
        z = jnp.concatenate([z16, z[:, 16:]], axis=1)
        return z * mask_ref[0]

    @pl.when(p == 0)
    def _():
        wb = w_ref[0].astype(jnp.bfloat16)
        wres_ref[:, rows, :] = wb
        z = jnp.dot(wb[0], u_ref[0], preferred_element_type=jnp.float32)
        z += jnp.dot(wb[1], u_ref[1], preferred_element_type=jnp.float32)
        z += jnp.dot(wb[2], u_ref[2], preferred_element_type=jnp.float32)
        xcur_ref[rows, :] = epilogue(z)

    @pl.when(p > 0)
    def _():
        z = jnp.dot(wres_ref[0, rows, :], u_ref[0],
                    preferred_element_type=jnp.float32)
        z += jnp.dot(wres_ref[1, rows, :], u_ref[1],
                     preferred_element_type=jnp.float32)
        z += jnp.dot(wres_ref[2, rows, :], u_ref[2],
                     preferred_element_type=jnp.float32)
        res = epilogue(z)

        @pl.when(p < nlast)
        def _():
            xcur_ref[rows, :] = res

        @pl.when(p == nlast)
        def _():
            o_ref[0] = res


def _fold(w1, w2, dcur):
    wcat = jnp.concatenate([w1, w2], axis=0)
    bm = wcat.reshape(32, 3, dcur).transpose(1, 2, 0)
    if dcur < 32:
        bm = jnp.pad(bm, ((0, 0), (0, 32 - dcur), (0, 0)))
    return bm


def kernel(W, x, mask, N_batch, fc1_w0, fc1_b0, fc2_w0, fc2_b0, fc1_w1, fc1_b1,
           fc2_w1, fc2_b1, fc1_w2, fc1_b2, fc2_w2, fc2_b2, fcl_w, fcl_b):
    bs, n = W.shape[0], W.shape[1]
    w_sep = jnp.transpose(W, (0, 3, 1, 2))
    x0 = jnp.pad(x, ((0, 0), (0, 0), (0, 32 - x.shape[-1])))

    wc3 = jnp.zeros((32, 96), jnp.float32).at[:2].set(fcl_w)
    b_all = jnp.stack([
        _fold(fc1_w0, fc2_w0, 8),
        _fold(fc1_w1, fc2_w1, 32),
        _fold(fc1_w2, fc2_w2, 32),
        wc3.reshape(32, 3, 32).transpose(1, 2, 0),
    ])
    bias_all = jnp.stack([
        jnp.concatenate([fc1_b0, fc2_b0]),
        jnp.concatenate([fc1_b1, fc2_b1]),
        jnp.concatenate([fc1_b2, fc2_b2]),
        jnp.zeros((32,), jnp.float32).at[:2].set(fcl_b),
    ])

    nt = n // _TN
    out = pl.pallas_call(
        _body,
        grid=(bs, 4, nt),
        in_specs=[
            pl.BlockSpec((1, 3, _TN, n),
                         lambda b, p, t: (b, 0,
                                          jnp.where(p == 0, t, nt - 1), 0)),
            pl.BlockSpec((1, n, 32), lambda b, p, t: (b, 0, 0)),
            pl.BlockSpec((4, 3, 32, 32), lambda b, p, t: (0, 0, 0, 0)),
            pl.BlockSpec((4, 32), lambda b, p, t: (0, 0)),
            pl.BlockSpec((1, _TN, 1), lambda b, p, t: (b, t, 0)),
        ],
        out_specs=pl.BlockSpec((1, _TN, 32), lambda b, p, t: (b, t, 0)),
        out_shape=jax.ShapeDtypeStruct((bs, n, 32), jnp.float32),
        scratch_shapes=[
            pltpu.VMEM((3, n, n), jnp.bfloat16),
            pltpu.VMEM((n, 32), jnp.float32),
            pltpu.VMEM((3, n, 32), jnp.bfloat16),
        ],
        compiler_params=pltpu.CompilerParams(
            dimension_semantics=("parallel", "arbitrary", "arbitrary"),
            vmem_limit_bytes=110 * 1024 * 1024),
    )(w_sep, x0, b_all, bias_all, mask)
    return out[:, :, :2]


# speedup vs baseline: 1.1010x; 1.1010x over previous
import jax
import jax.numpy as jnp
from jax.experimental import pallas as pl
from jax.experimental.pallas import tpu as pltpu

_TN = 512
_N = 2048


def _body(w_ref, x_ref, b_ref, bias_ref, mask_ref, o_ref,
          wres_ref, xcur_ref, u_ref):
    b = pl.program_id(0)
    p = pl.program_id(1)
    t = pl.program_id(2)
    nlast = pl.num_programs(1) - 1

    @pl.when(t == 0)
    def _():
        @pl.when(p == 0)
        def _():
            xb = x_ref[0]
            for jj in range(3):
                u_ref[pl.ds(jj * _N, _N), :] = jnp.dot(
                    xb, b_ref[0, jj], preferred_element_type=jnp.float32
                ).astype(jnp.bfloat16)

        @pl.when(p > 0)
        def _():
            xb = xcur_ref[...]
            for jj in range(3):
                u_ref[pl.ds(jj * _N, _N), :] = jnp.dot(
                    xb, b_ref[p, jj], preferred_element_type=jnp.float32
                ).astype(jnp.bfloat16)

    rows = pl.ds(t * _TN, _TN)

    def epilogue(z):
        z = z + bias_ref[p][None]
        z16 = jnp.where(p < nlast, jnp.maximum(z[:, :16], 0.0), z[:, :16])
        z = jnp.concatenate([z16, z[:, 16:]], axis=1)
        return z * mask_ref[0]

    @pl.when(p == 0)
    def _():
        wb = w_ref[0].astype(jnp.bfloat16)
        for jj in range(3):
            wres_ref[rows, pl.ds(jj * _N, _N)] = wb[jj]
        z = jnp.dot(wb[0], u_ref[pl.ds(0, _N), :],
                    preferred_element_type=jnp.float32)
        z += jnp.dot(wb[1], u_ref[pl.ds(_N, _N), :],
                     preferred_element_type=jnp.float32)
        z += jnp.dot(wb[2], u_ref[pl.ds(2 * _N, _N), :],
                     preferred_element_type=jnp.float32)
        xcur_ref[rows, :] = epilogue(z)

    @pl.when(p > 0)
    def _():
        z = jnp.dot(wres_ref[rows, :], u_ref[...],
                    preferred_element_type=jnp.float32)
        res = epilogue(z)

        @pl.when(p < nlast)
        def _():
            xcur_ref[rows, :] = res

        @pl.when(p == nlast)
        def _():
            o_ref[0] = res


def _fold(w1, w2, dcur):
    wcat = jnp.concatenate([w1, w2], axis=0)
    bm = wcat.reshape(32, 3, dcur).transpose(1, 2, 0)
    if dcur < 32:
        bm = jnp.pad(bm, ((0, 0), (0, 32 - dcur), (0, 0)))
    return bm


def kernel(W, x, mask, N_batch, fc1_w0, fc1_b0, fc2_w0, fc2_b0, fc1_w1, fc1_b1,
           fc2_w1, fc2_b1, fc1_w2, fc1_b2, fc2_w2, fc2_b2, fcl_w, fcl_b):
    bs, n = W.shape[0], W.shape[1]
    w_sep = jnp.transpose(W, (0, 3, 1, 2))
    x0 = jnp.pad(x, ((0, 0), (0, 0), (0, 32 - x.shape[-1])))

    wc3 = jnp.zeros((32, 96), jnp.float32).at[:2].set(fcl_w)
    b_all = jnp.stack([
        _fold(fc1_w0, fc2_w0, 8),
        _fold(fc1_w1, fc2_w1, 32),
        _fold(fc1_w2, fc2_w2, 32),
        wc3.reshape(32, 3, 32).transpose(1, 2, 0),
    ])
    bias_all = jnp.stack([
        jnp.concatenate([fc1_b0, fc2_b0]),
        jnp.concatenate([fc1_b1, fc2_b1]),
        jnp.concatenate([fc1_b2, fc2_b2]),
        jnp.zeros((32,), jnp.float32).at[:2].set(fcl_b),
    ])

    nt = n // _TN
    out = pl.pallas_call(
        _body,
        grid=(bs, 4, nt),
        in_specs=[
            pl.BlockSpec((1, 3, _TN, n),
                         lambda b, p, t: (b, 0,
                                          jnp.where(p == 0, t, nt - 1), 0)),
            pl.BlockSpec((1, n, 32), lambda b, p, t: (b, 0, 0)),
            pl.BlockSpec((4, 3, 32, 32), lambda b, p, t: (0, 0, 0, 0)),
            pl.BlockSpec((4, 32), lambda b, p, t: (0, 0)),
            pl.BlockSpec((1, _TN, 1), lambda b, p, t: (b, t, 0)),
        ],
        out_specs=pl.BlockSpec((1, _TN, 32), lambda b, p, t: (b, t, 0)),
        out_shape=jax.ShapeDtypeStruct((bs, n, 32), jnp.float32),
        scratch_shapes=[
            pltpu.VMEM((n, 3 * n), jnp.bfloat16),
            pltpu.VMEM((n, 32), jnp.float32),
            pltpu.VMEM((3 * n, 32), jnp.bfloat16),
        ],
        compiler_params=pltpu.CompilerParams(
            dimension_semantics=("parallel", "arbitrary", "arbitrary"),
            vmem_limit_bytes=110 * 1024 * 1024),
    )(w_sep, x0, b_all, bias_all, mask)
    return out[:, :, :2]
